# baseline (speedup 1.0000x reference)
import functools

import jax
import jax.numpy as jnp
from jax.experimental import pallas as pl
from jax.experimental.pallas import tpu as pltpu

_R = 5
_BR = 40


def _graph_conv_kernel(*refs):
    a_refs = refs[:2 * _R]
    x_ref, xi_ref, w0t_ref, w1t_ref, b_ref, o_ref = refs[2 * _R:]
    x = x_ref[:]
    w0t = w0t_ref[:]
    w1t = w1t_ref[:]
    bias = b_ref[:]
    for r in range(_R):
        a0 = a_refs[r][:].astype(jnp.bfloat16)
        a1 = a_refs[_R + r][:].astype(jnp.bfloat16)
        p0 = jnp.dot(a0, x, preferred_element_type=jnp.float32)
        p1 = jnp.dot(a1, x, preferred_element_type=jnp.float32)
        xi = xi_ref[r * _BR:(r + 1) * _BR, :]
        o_ref[r * _BR:(r + 1) * _BR, :] = (
            jnp.dot(p0 + xi, w0t, preferred_element_type=jnp.float32)
            + jnp.dot(p1 + xi, w1t, preferred_element_type=jnp.float32)
            + bias
        )


@jax.jit
def _graph_conv(x0, a0, a1, w0t, w1t, b2d):
    n, d = x0.shape
    out = w0t.shape[1]
    bn = _R * _BR
    a_specs = [
        pl.BlockSpec((_BR, n), functools.partial(
            lambda i, r=0: (i * _R + r, 0), r=r))
        for r in range(_R)
    ]
    return pl.pallas_call(
        _graph_conv_kernel,
        grid=(n // bn,),
        in_specs=(
            a_specs + a_specs +
            [
                pl.BlockSpec((n, d), lambda i: (0, 0)),
                pl.BlockSpec((bn, d), lambda i: (i, 0)),
                pl.BlockSpec((d, out), lambda i: (0, 0)),
                pl.BlockSpec((d, out), lambda i: (0, 0)),
                pl.BlockSpec((1, out), lambda i: (0, 0)),
            ]
        ),
        out_specs=pl.BlockSpec((bn, out), lambda i: (i, 0)),
        out_shape=jax.ShapeDtypeStruct((n, out), jnp.float32),
        compiler_params=pltpu.CompilerParams(
            dimension_semantics=("parallel",),
        ),
    )(*([a0] * _R), *([a1] * _R), x0, x0, w0t, w1t, b2d)


def kernel(inputs, supports, W, b):
    bsz, n, d = inputs.shape
    s = supports.shape[0]
    out_dim = W.shape[0]
    x0 = jnp.transpose(inputs, (1, 2, 0)).reshape(n, d * bsz)
    w0t = jnp.transpose(W[:, 0::s])
    w1t = jnp.transpose(W[:, 1::s])
    b2d = b.reshape(1, out_dim)

    res = _graph_conv(x0.astype(jnp.bfloat16), supports[0], supports[1],
                      w0t, w1t, b2d)
    return res.reshape(bsz, n, out_dim)


# speedup vs baseline: 2.8310x; 2.8310x over previous
import functools

import jax
import jax.numpy as jnp
from jax.experimental import pallas as pl
from jax.experimental.pallas import tpu as pltpu

_R = 5
_BR = 40


def _graph_conv_kernel(*refs):
    a_refs = refs[:2 * _R]
    x_ref, xi_ref, w0t_ref, w1t_ref, b_ref, o_ref = refs[2 * _R:]
    x = x_ref[:]
    w0t = w0t_ref[:]
    w1t = w1t_ref[:]
    bias = b_ref[:]
    for r in range(_R):
        a0 = a_refs[r][0].astype(jnp.bfloat16)
        a1 = a_refs[_R + r][0].astype(jnp.bfloat16)
        p0 = jnp.dot(a0, x, preferred_element_type=jnp.float32)
        p1 = jnp.dot(a1, x, preferred_element_type=jnp.float32)
        xi = xi_ref[r * _BR:(r + 1) * _BR, :]
        o_ref[r * _BR:(r + 1) * _BR, :] = (
            jnp.dot(p0 + xi, w0t, preferred_element_type=jnp.float32)
            + jnp.dot(p1 + xi, w1t, preferred_element_type=jnp.float32)
            + bias
        )


@jax.jit
def _graph_conv(x0_bf16, x0, supports, w0t, w1t, b2d):
    n, d = x0.shape
    out = w0t.shape[1]
    bn = _R * _BR
    a_specs = [
        pl.BlockSpec((1, _BR, n), functools.partial(
            lambda i, s=0, r=0: (s, i * _R + r, 0), s=s, r=r))
        for s in range(2)
        for r in range(_R)
    ]
    return pl.pallas_call(
        _graph_conv_kernel,
        grid=(n // bn,),
        in_specs=(
            a_specs +
            [
                pl.BlockSpec((n, d), lambda i: (0, 0)),
                pl.BlockSpec((bn, d), lambda i: (i, 0)),
                pl.BlockSpec((d, out), lambda i: (0, 0)),
                pl.BlockSpec((d, out), lambda i: (0, 0)),
                pl.BlockSpec((1, out), lambda i: (0, 0)),
            ]
        ),
        out_specs=pl.BlockSpec((bn, out), lambda i: (i, 0)),
        out_shape=jax.ShapeDtypeStruct((n, out), jnp.float32),
        compiler_params=pltpu.CompilerParams(
            dimension_semantics=("parallel",),
        ),
    )(*([supports] * (2 * _R)), x0_bf16, x0, w0t, w1t, b2d)


def kernel(inputs, supports, W, b):
    bsz, n, d = inputs.shape
    s = supports.shape[0]
    out_dim = W.shape[0]
    x0 = jnp.transpose(inputs, (1, 2, 0)).reshape(n, d * bsz)
    w0t = jnp.transpose(W[:, 0::s])
    w1t = jnp.transpose(W[:, 1::s])
    b2d = b.reshape(1, out_dim)

    res = _graph_conv(x0.astype(jnp.bfloat16), x0, supports, w0t, w1t, b2d)
    return res.reshape(bsz, n, out_dim)


# speedup vs baseline: 3.0282x; 1.0696x over previous
import functools

import jax
import jax.numpy as jnp
from jax.experimental import pallas as pl
from jax.experimental.pallas import tpu as pltpu

_R = 1
_BR = 200


def _graph_conv_kernel(*refs):
    a_refs = refs[:2 * _R]
    x_ref, xi_ref, w0t_ref, w1t_ref, b_ref, o_ref = refs[2 * _R:]
    x = x_ref[:]
    w0t = w0t_ref[:]
    w1t = w1t_ref[:]
    bias = b_ref[:]
    for r in range(_R):
        a0 = a_refs[r][0].astype(jnp.bfloat16)
        a1 = a_refs[_R + r][0].astype(jnp.bfloat16)
        p0 = jnp.dot(a0, x, preferred_element_type=jnp.float32)
        p1 = jnp.dot(a1, x, preferred_element_type=jnp.float32)
        xi = xi_ref[r * _BR:(r + 1) * _BR, :]
        o_ref[r * _BR:(r + 1) * _BR, :] = (
            jnp.dot(p0 + xi, w0t, preferred_element_type=jnp.float32)
            + jnp.dot(p1 + xi, w1t, preferred_element_type=jnp.float32)
            + bias
        )


@jax.jit
def _graph_conv(x0_bf16, x0, supports, w0t, w1t, b2d):
    n, d = x0.shape
    out = w0t.shape[1]
    bn = _R * _BR
    a_specs = [
        pl.BlockSpec((1, _BR, n), functools.partial(
            lambda i, s=0, r=0: (s, i * _R + r, 0), s=s, r=r))
        for s in range(2)
        for r in range(_R)
    ]
    return pl.pallas_call(
        _graph_conv_kernel,
        grid=(n // bn,),
        in_specs=(
            a_specs +
            [
                pl.BlockSpec((n, d), lambda i: (0, 0)),
                pl.BlockSpec((bn, d), lambda i: (i, 0)),
                pl.BlockSpec((d, out), lambda i: (0, 0)),
                pl.BlockSpec((d, out), lambda i: (0, 0)),
                pl.BlockSpec((1, out), lambda i: (0, 0)),
            ]
        ),
        out_specs=pl.BlockSpec((bn, out), lambda i: (i, 0)),
        out_shape=jax.ShapeDtypeStruct((n, out), jnp.float32),
        compiler_params=pltpu.CompilerParams(
            dimension_semantics=("parallel",),
        ),
    )(*([supports] * (2 * _R)), x0_bf16, x0, w0t, w1t, b2d)


def kernel(inputs, supports, W, b):
    bsz, n, d = inputs.shape
    s = supports.shape[0]
    out_dim = W.shape[0]
    x0 = jnp.transpose(inputs, (1, 2, 0)).reshape(n, d * bsz)
    w0t = jnp.transpose(W[:, 0::s])
    w1t = jnp.transpose(W[:, 1::s])
    b2d = b.reshape(1, out_dim)

    res = _graph_conv(x0.astype(jnp.bfloat16), x0, supports, w0t, w1t, b2d)
    return res.reshape(bsz, n, out_dim)
